# baseline (speedup 1.0000x reference)
import numpy as np

import jax
import jax.numpy as jnp
from jax import lax
from jax.experimental import pallas as pl

D_MODEL = 128
B, S, F = 4096, 200, 10
BB = 16


def _make_pe(d_model, max_len):
    position = np.arange(max_len, dtype=np.float32)[:, None]
    div_term = np.exp(np.arange(0, d_model, 2, dtype=np.float32)
                      * -(np.log(10000.0) / d_model))
    pe = np.zeros((max_len, d_model), dtype=np.float32)
    pe[:, 0::2] = np.sin(position * div_term)
    pe[:, 1::2] = np.cos(position * div_term)
    return pe


_PE_NP = _make_pe(D_MODEL, S)


def _tc_body(x_ref, hour_ref, quarter_ref, wsin_ref, wcos_ref,
             bsin_ref, bcos_ref, pe_ref, out_ref):
    xb = x_ref[...]
    ids = (xb[:, :, 0].astype(jnp.int32) * 4 + xb[:, :, 1].astype(jnp.int32))
    a = xb[:, :, 4] + xb[:, :, 5]
    c = xb[:, :, 5] + xb[:, :, 6]
    t96 = ((hour_ref[...][:, None, :] + quarter_ref[...][None, :, :])
           .reshape(96, D_MODEL)
           + 2.0 * (bsin_ref[...] + bcos_ref[...]))
    idsf = ids.reshape(BB * S, 1)
    onehot = (idsf == lax.broadcasted_iota(jnp.int32, (BB * S, 96), 1)
              ).astype(jnp.float32)
    g = lax.dot_general(onehot, t96, (((1,), (0,)), ((), ())),
                        preferred_element_type=jnp.float32)
    w_s = wsin_ref[...]
    w_c = wcos_ref[...]
    out_ref[...] = (g.reshape(BB, S, D_MODEL) + pe_ref[...][None]
                    + a[:, :, None] * w_s[None]
                    + c[:, :, None] * w_c[None])


def kernel(x, hour_table, quarter_table, W_sin, b_sin, W_cos, b_cos):
    pe = jnp.asarray(_PE_NP)
    wsin = W_sin.reshape(1, D_MODEL)
    wcos = W_cos.reshape(1, D_MODEL)
    bsin = b_sin.reshape(1, D_MODEL)
    bcos = b_cos.reshape(1, D_MODEL)
    f32 = jnp.float32
    full = lambda i: (0, 0)
    out = pl.pallas_call(
        _tc_body,
        grid=(B // BB,),
        in_specs=[
            pl.BlockSpec((BB, S, F), lambda i: (i, 0, 0)),
            pl.BlockSpec((24, D_MODEL), full),
            pl.BlockSpec((4, D_MODEL), full),
            pl.BlockSpec((1, D_MODEL), full),
            pl.BlockSpec((1, D_MODEL), full),
            pl.BlockSpec((1, D_MODEL), full),
            pl.BlockSpec((1, D_MODEL), full),
            pl.BlockSpec((S, D_MODEL), full),
        ],
        out_specs=pl.BlockSpec((BB, S, D_MODEL), lambda i: (i, 0, 0)),
        out_shape=jax.ShapeDtypeStruct((B, S, D_MODEL), f32),
    )(x, hour_table, quarter_table, wsin, wcos, bsin, bcos, pe)
    return out
